# Initial kernel scaffold; baseline (speedup 1.0000x reference)
import functools

import jax
import jax.numpy as jnp
from jax import lax
from jax.experimental import pallas as pl
from jax.experimental.pallas import tpu as pltpu
from jax.experimental.pallas import tpu_sc as plsc

N = 10000
E = 320000
D = 128
NC = 40
NP = 10240
TRASH = N
NCORE = 2
NSUB = 16
NW = NCORE * NSUB
EPW = E // NW
CH = 80
NCH = EPW // CH
ROWS_PER_TILE = NP // NSUB

_mesh = plsc.VectorSubcoreMesh(core_axis_name="c", subcore_axis_name="s")

_f32 = jnp.float32
_i32 = jnp.int32


@functools.partial(
    pl.kernel,
    out_type=[
        jax.ShapeDtypeStruct((NP, 16), _f32),
        jax.ShapeDtypeStruct((NP, 16), _f32),
        jax.ShapeDtypeStruct((E // CH, CH), _i32),
    ],
    mesh=_mesh,
    scratch_types=[
        pltpu.VMEM((E // CH // NW, CH), _i32),
        pltpu.VMEM((E // CH // NW, CH), _i32),
        pltpu.VMEM((E // CH // NW, CH), _i32),
        pltpu.VMEM((CH, 16), _f32),
        pltpu.VMEM((128, 16), _f32),
        pltpu.VMEM_SHARED((NP, 16), _f32),
    ],
)
def _sc_prep(src_hbm, dst_hbm, h0_hbm, h1_hbm, dstm_hbm,
             src_v, dst_v, dstm_v, ones_v, zero_v, hist_sh):
    cid = lax.axis_index("c")
    sid = lax.axis_index("s")
    wid = sid * NCORE + cid
    base = wid * NCH

    lane = lax.iota(_i32, 16)
    e0 = jnp.where(lane == 0, 1.0, 0.0).astype(_f32)
    z16 = jnp.zeros((16,), _f32)

    @pl.loop(0, CH)
    def _(r):
        ones_v[r, :] = e0

    @pl.loop(0, 128)
    def _(r):
        zero_v[r, :] = z16

    @pl.loop(0, ROWS_PER_TILE // 128)
    def _(k):
        pltpu.sync_copy(zero_v, hist_sh.at[pl.ds(sid * ROWS_PER_TILE + k * 128, 128)])

    pltpu.sync_copy(src_hbm.at[pl.ds(base, NCH)], src_v)
    pltpu.sync_copy(dst_hbm.at[pl.ds(base, NCH)], dst_v)

    @pl.loop(0, NCH)
    def _(r):
        @pl.loop(0, CH // 16)
        def _(c):
            s = src_v[r, pl.ds(c * 16, 16)]
            dv = dst_v[r, pl.ds(c * 16, 16)]
            dstm_v[r, pl.ds(c * 16, 16)] = jnp.where(s == dv, TRASH, dv)

    pltpu.sync_copy(dstm_v, dstm_hbm.at[pl.ds(base, NCH)])

    plsc.subcore_barrier()

    @pl.loop(0, NCH)
    def _(r):
        pltpu.sync_copy(ones_v, hist_sh.at[dstm_v.at[r]], add=True)

    plsc.subcore_barrier()

    @pl.when(cid == 0)
    def _():
        pltpu.sync_copy(hist_sh.at[pl.ds(sid * ROWS_PER_TILE, ROWS_PER_TILE)],
                        h0_hbm.at[pl.ds(sid * ROWS_PER_TILE, ROWS_PER_TILE)])

    @pl.when(cid == 1)
    def _():
        pltpu.sync_copy(hist_sh.at[pl.ds(sid * ROWS_PER_TILE, ROWS_PER_TILE)],
                        h1_hbm.at[pl.ds(sid * ROWS_PER_TILE, ROWS_PER_TILE)])


@functools.partial(
    pl.kernel,
    out_type=[
        jax.ShapeDtypeStruct((NP, D), _f32),
        jax.ShapeDtypeStruct((NP, D), _f32),
    ],
    mesh=_mesh,
    scratch_types=[
        pltpu.VMEM((E // CH // NW, CH), _i32),
        pltpu.VMEM((E // CH // NW, CH), _i32),
        pltpu.VMEM((CH, D), _f32),
        pltpu.VMEM((CH, D), _f32),
        pltpu.VMEM((64, D), _f32),
        pltpu.VMEM_SHARED((NP, D), _f32),
        pltpu.SemaphoreType.DMA,
        pltpu.SemaphoreType.DMA,
    ],
)
def _sc_agg(hs_hbm, src_hbm, dstm_hbm, a0_hbm, a1_hbm,
            src_v, dstm_v, rows0, rows1, zero_v, acc_sh, gsem0, gsem1):
    cid = lax.axis_index("c")
    sid = lax.axis_index("s")
    wid = sid * NCORE + cid
    base = wid * NCH

    z16 = jnp.zeros((16,), _f32)

    @pl.loop(0, 64)
    def _(r):
        @pl.loop(0, D // 16)
        def _(c):
            zero_v[r, pl.ds(c * 16, 16)] = z16

    @pl.loop(0, ROWS_PER_TILE // 64)
    def _(k):
        pltpu.sync_copy(zero_v, acc_sh.at[pl.ds(sid * ROWS_PER_TILE + k * 64, 64)])

    pltpu.sync_copy(src_hbm.at[pl.ds(base, NCH)], src_v)
    pltpu.sync_copy(dstm_hbm.at[pl.ds(base, NCH)], dstm_v)

    plsc.subcore_barrier()

    pltpu.async_copy(hs_hbm.at[src_v.at[0]], rows0, gsem0)

    @pl.loop(0, NCH - 1, step=2)
    def _(i):
        pltpu.async_copy(hs_hbm.at[src_v.at[i + 1]], rows1, gsem1)
        pltpu.make_async_copy(hs_hbm.at[src_v.at[0]], rows0, gsem0).wait()
        pltpu.sync_copy(rows0, acc_sh.at[dstm_v.at[i]], add=True)
        pltpu.async_copy(hs_hbm.at[src_v.at[i + 2]], rows0, gsem0)
        pltpu.make_async_copy(hs_hbm.at[src_v.at[0]], rows1, gsem1).wait()
        pltpu.sync_copy(rows1, acc_sh.at[dstm_v.at[i + 1]], add=True)

    pltpu.make_async_copy(hs_hbm.at[src_v.at[0]], rows0, gsem0).wait()
    pltpu.sync_copy(rows0, acc_sh.at[dstm_v.at[NCH - 1]], add=True)

    plsc.subcore_barrier()

    @pl.when(cid == 0)
    def _():
        pltpu.sync_copy(acc_sh.at[pl.ds(sid * ROWS_PER_TILE, ROWS_PER_TILE)],
                        a0_hbm.at[pl.ds(sid * ROWS_PER_TILE, ROWS_PER_TILE)])

    @pl.when(cid == 1)
    def _():
        pltpu.sync_copy(acc_sh.at[pl.ds(sid * ROWS_PER_TILE, ROWS_PER_TILE)],
                        a1_hbm.at[pl.ds(sid * ROWS_PER_TILE, ROWS_PER_TILE)])


_BM = 1024


def _dot(a, b):
    return jnp.dot(a, b, preferred_element_type=_f32,
                   precision=jax.lax.Precision.HIGHEST)


def _mm_body(x_ref, w_ref, o_ref):
    o_ref[...] = _dot(x_ref[...], w_ref[...])


def _tc_matmul(x, w):
    return pl.pallas_call(
        _mm_body,
        grid=(NP // _BM,),
        in_specs=[pl.BlockSpec((_BM, D), lambda i: (i, 0)),
                  pl.BlockSpec((D, D), lambda i: (0, 0))],
        out_specs=pl.BlockSpec((_BM, D), lambda i: (i, 0)),
        out_shape=jax.ShapeDtypeStruct((NP, D), _f32),
    )(x, w)


def _scale_body(h0_ref, h1_ref, u_ref, dinv_ref, hs_ref):
    deg = h0_ref[:, 0:1] + h1_ref[:, 0:1] + 1.0
    dinv = lax.rsqrt(deg)
    dinv_ref[...] = dinv
    hs_ref[...] = dinv * u_ref[...]


def _tc_scale(h0, h1, u):
    return pl.pallas_call(
        _scale_body,
        grid=(NP // _BM,),
        in_specs=[pl.BlockSpec((_BM, 16), lambda i: (i, 0)),
                  pl.BlockSpec((_BM, 16), lambda i: (i, 0)),
                  pl.BlockSpec((_BM, D), lambda i: (i, 0))],
        out_specs=[pl.BlockSpec((_BM, 1), lambda i: (i, 0)),
                   pl.BlockSpec((_BM, D), lambda i: (i, 0))],
        out_shape=[jax.ShapeDtypeStruct((NP, 1), _f32),
                   jax.ShapeDtypeStruct((NP, D), _f32)],
    )(h0, h1, u)


def _layer_body(a0_ref, a1_ref, hs_ref, dinv_ref, b_ref, w_ref, o_ref):
    dinv = dinv_ref[...]
    z = dinv * (a0_ref[...] + a1_ref[...] + hs_ref[...]) + b_ref[...]
    z = jnp.maximum(z, 0.0)
    o_ref[...] = dinv * _dot(z, w_ref[...])


def _tc_layer(a0, a1, hs, dinv, b, w):
    return pl.pallas_call(
        _layer_body,
        grid=(NP // _BM,),
        in_specs=[pl.BlockSpec((_BM, D), lambda i: (i, 0)),
                  pl.BlockSpec((_BM, D), lambda i: (i, 0)),
                  pl.BlockSpec((_BM, D), lambda i: (i, 0)),
                  pl.BlockSpec((_BM, 1), lambda i: (i, 0)),
                  pl.BlockSpec((1, D), lambda i: (0, 0)),
                  pl.BlockSpec((D, D), lambda i: (0, 0))],
        out_specs=pl.BlockSpec((_BM, D), lambda i: (i, 0)),
        out_shape=jax.ShapeDtypeStruct((NP, D), _f32),
    )(a0, a1, hs, dinv, b, w)


def _out_body(a0_ref, a1_ref, hs_ref, dinv_ref, b_ref, wc_ref, bc_ref,
              lo_ref, lp_ref):
    dinv = dinv_ref[...]
    z = dinv * (a0_ref[...] + a1_ref[...] + hs_ref[...]) + b_ref[...]
    logits = _dot(z, wc_ref[...]) + bc_ref[...]
    m = jnp.max(logits, axis=1, keepdims=True)
    lse = m + jnp.log(jnp.sum(jnp.exp(logits - m), axis=1, keepdims=True))
    lo_ref[...] = logits
    lp_ref[...] = logits - lse


def _tc_out(a0, a1, hs, dinv, b, wc, bc):
    return pl.pallas_call(
        _out_body,
        grid=(NP // _BM,),
        in_specs=[pl.BlockSpec((_BM, D), lambda i: (i, 0)),
                  pl.BlockSpec((_BM, D), lambda i: (i, 0)),
                  pl.BlockSpec((_BM, D), lambda i: (i, 0)),
                  pl.BlockSpec((_BM, 1), lambda i: (i, 0)),
                  pl.BlockSpec((1, D), lambda i: (0, 0)),
                  pl.BlockSpec((D, D), lambda i: (0, 0)),
                  pl.BlockSpec((1, D), lambda i: (0, 0))],
        out_specs=[pl.BlockSpec((_BM, D), lambda i: (i, 0)),
                   pl.BlockSpec((_BM, D), lambda i: (i, 0))],
        out_shape=[jax.ShapeDtypeStruct((NP, D), _f32),
                   jax.ShapeDtypeStruct((NP, D), _f32)],
    )(a0, a1, hs, dinv, b, wc, bc)


def kernel(x, edge_index, W1, b1, W2, b2, Wc, bc):
    src = edge_index[0].reshape(E // CH, CH)
    dst = edge_index[1].reshape(E // CH, CH)
    x_pad = jnp.pad(x, ((0, NP - N), (0, 0)))

    u1 = _tc_matmul(x_pad, W1)
    h0, h1, dstm = _sc_prep(src, dst)

    dinv, hs1 = _tc_scale(h0, h1, u1)
    a0, a1 = _sc_agg(hs1, src, dstm)
    hs2 = _tc_layer(a0, a1, hs1, dinv, b1.reshape(1, D), W2)
    c0, c1 = _sc_agg(hs2, src, dstm)

    wcp = jnp.pad(Wc, ((0, 0), (0, D - NC)))
    bcp = jnp.pad(bc, (0, D - NC), constant_values=-1e30).reshape(1, D)
    logits_pad, logp_pad = _tc_out(c0, c1, hs2, dinv, b2.reshape(1, D), wcp, bcp)
    return logits_pad[:N, :NC], logp_pad[:N, :NC]


# speedup vs baseline: 7.6150x; 7.6150x over previous
import functools

import jax
import jax.numpy as jnp
from jax import lax
from jax.experimental import pallas as pl
from jax.experimental.pallas import tpu as pltpu
from jax.experimental.pallas import tpu_sc as plsc

N = 10000
E = 320000
D = 128
NC = 40
NP = 10240
TRASH = N
NCORE = 2
NSUB = 16
NW = NCORE * NSUB
CH = 128
EROWS = 2560
NCH = EROWS // NW
SEG = 16
ROWS_PER_TILE = NP // NSUB

_mesh = plsc.VectorSubcoreMesh(core_axis_name="c", subcore_axis_name="s")

_f32 = jnp.float32
_i32 = jnp.int32


@functools.partial(
    pl.kernel,
    out_type=[
        jax.ShapeDtypeStruct((NP, D), _f32),
        jax.ShapeDtypeStruct((NP, D), _f32),
        jax.ShapeDtypeStruct((EROWS, CH), _i32),
    ],
    mesh=_mesh,
    scratch_types=[
        pltpu.VMEM((SEG, CH), _i32),
        pltpu.VMEM((SEG, CH), _i32),
        pltpu.VMEM((SEG, CH), _i32),
        pltpu.VMEM((CH, D), _f32),
        pltpu.VMEM_SHARED((NP, D), _f32),
    ],
)
def _sc_prep(src_hbm, dst_hbm, h0_hbm, h1_hbm, dstm_hbm,
             src_v, dst_v, dstm_v, buf_v, hist_sh):
    cid = lax.axis_index("c")
    sid = lax.axis_index("s")
    wid = sid * NCORE + cid
    base = wid * NCH

    z16 = jnp.zeros((16,), _f32)
    o16 = jnp.ones((16,), _f32)

    @pl.loop(0, CH)
    def _(r):
        @pl.loop(0, D // 16)
        def _(c):
            buf_v[r, pl.ds(c * 16, 16)] = z16

    @pl.loop(0, ROWS_PER_TILE // CH)
    def _(k):
        pltpu.sync_copy(buf_v, hist_sh.at[pl.ds(sid * ROWS_PER_TILE + k * CH, CH)])

    @pl.loop(0, CH)
    def _(r):
        @pl.loop(0, D // 16)
        def _(c):
            buf_v[r, pl.ds(c * 16, 16)] = o16

    plsc.subcore_barrier()

    for g in range(NCH // SEG):
        gb = base + g * SEG
        pltpu.sync_copy(src_hbm.at[pl.ds(gb, SEG)], src_v)
        pltpu.sync_copy(dst_hbm.at[pl.ds(gb, SEG)], dst_v)

        @pl.loop(0, SEG)
        def _(r):
            @pl.loop(0, CH // 16)
            def _(c):
                s = src_v[r, pl.ds(c * 16, 16)]
                dv = dst_v[r, pl.ds(c * 16, 16)]
                dstm_v[r, pl.ds(c * 16, 16)] = jnp.where(s == dv, TRASH, dv)

        pltpu.sync_copy(dstm_v, dstm_hbm.at[pl.ds(gb, SEG)])

        @pl.loop(0, SEG)
        def _(r):
            pltpu.sync_copy(buf_v, hist_sh.at[dstm_v.at[r]], add=True)

    plsc.subcore_barrier()

    @pl.when(cid == 0)
    def _():
        pltpu.sync_copy(hist_sh.at[pl.ds(sid * ROWS_PER_TILE, ROWS_PER_TILE)],
                        h0_hbm.at[pl.ds(sid * ROWS_PER_TILE, ROWS_PER_TILE)])

    @pl.when(cid == 1)
    def _():
        pltpu.sync_copy(hist_sh.at[pl.ds(sid * ROWS_PER_TILE, ROWS_PER_TILE)],
                        h1_hbm.at[pl.ds(sid * ROWS_PER_TILE, ROWS_PER_TILE)])


@functools.partial(
    pl.kernel,
    out_type=[
        jax.ShapeDtypeStruct((NP, D), _f32),
        jax.ShapeDtypeStruct((NP, D), _f32),
    ],
    mesh=_mesh,
    scratch_types=[
        pltpu.VMEM((SEG, CH), _i32),
        pltpu.VMEM((SEG, CH), _i32),
        pltpu.VMEM((CH, D), _f32),
        pltpu.VMEM_SHARED((NP, D), _f32),
    ],
)
def _sc_agg(hs_hbm, src_hbm, dstm_hbm, a0_hbm, a1_hbm,
            src_v, dstm_v, rows0, acc_sh):
    cid = lax.axis_index("c")
    sid = lax.axis_index("s")
    wid = sid * NCORE + cid
    base = wid * NCH

    z16 = jnp.zeros((16,), _f32)

    @pl.loop(0, CH)
    def _(r):
        @pl.loop(0, D // 16)
        def _(c):
            rows0[r, pl.ds(c * 16, 16)] = z16

    @pl.loop(0, ROWS_PER_TILE // CH)
    def _(k):
        pltpu.sync_copy(rows0, acc_sh.at[pl.ds(sid * ROWS_PER_TILE + k * CH, CH)])

    plsc.subcore_barrier()

    for g in range(NCH // SEG):
        gb = base + g * SEG
        pltpu.sync_copy(src_hbm.at[pl.ds(gb, SEG)], src_v)
        pltpu.sync_copy(dstm_hbm.at[pl.ds(gb, SEG)], dstm_v)

        @pl.loop(0, SEG)
        def _(r):
            pltpu.sync_copy(hs_hbm.at[src_v.at[r]], rows0)
            pltpu.sync_copy(rows0, acc_sh.at[dstm_v.at[r]], add=True)

    plsc.subcore_barrier()

    @pl.when(cid == 0)
    def _():
        pltpu.sync_copy(acc_sh.at[pl.ds(sid * ROWS_PER_TILE, ROWS_PER_TILE)],
                        a0_hbm.at[pl.ds(sid * ROWS_PER_TILE, ROWS_PER_TILE)])

    @pl.when(cid == 1)
    def _():
        pltpu.sync_copy(acc_sh.at[pl.ds(sid * ROWS_PER_TILE, ROWS_PER_TILE)],
                        a1_hbm.at[pl.ds(sid * ROWS_PER_TILE, ROWS_PER_TILE)])


_BM = 1024


def _dot(a, b):
    return jnp.dot(a, b, preferred_element_type=_f32,
                   precision=jax.lax.Precision.HIGHEST)


def _mm_body(x_ref, w_ref, o_ref):
    o_ref[...] = _dot(x_ref[...], w_ref[...])


def _tc_matmul(x, w):
    return pl.pallas_call(
        _mm_body,
        grid=(NP // _BM,),
        in_specs=[pl.BlockSpec((_BM, D), lambda i: (i, 0)),
                  pl.BlockSpec((D, D), lambda i: (0, 0))],
        out_specs=pl.BlockSpec((_BM, D), lambda i: (i, 0)),
        out_shape=jax.ShapeDtypeStruct((NP, D), _f32),
    )(x, w)


def _scale_body(h0_ref, h1_ref, u_ref, dinv_ref, hs_ref):
    deg = h0_ref[...] + h1_ref[...] + 1.0
    dinv = lax.rsqrt(deg)
    dinv_ref[...] = dinv
    hs_ref[...] = dinv * u_ref[...]


def _tc_scale(h0, h1, u):
    return pl.pallas_call(
        _scale_body,
        grid=(NP // _BM,),
        in_specs=[pl.BlockSpec((_BM, D), lambda i: (i, 0)),
                  pl.BlockSpec((_BM, D), lambda i: (i, 0)),
                  pl.BlockSpec((_BM, D), lambda i: (i, 0))],
        out_specs=[pl.BlockSpec((_BM, D), lambda i: (i, 0)),
                   pl.BlockSpec((_BM, D), lambda i: (i, 0))],
        out_shape=[jax.ShapeDtypeStruct((NP, D), _f32),
                   jax.ShapeDtypeStruct((NP, D), _f32)],
    )(h0, h1, u)


def _layer_body(a0_ref, a1_ref, hs_ref, dinv_ref, b_ref, w_ref, o_ref):
    dinv = dinv_ref[...]
    z = dinv * (a0_ref[...] + a1_ref[...] + hs_ref[...]) + b_ref[...]
    z = jnp.maximum(z, 0.0)
    o_ref[...] = dinv * _dot(z, w_ref[...])


def _tc_layer(a0, a1, hs, dinv, b, w):
    return pl.pallas_call(
        _layer_body,
        grid=(NP // _BM,),
        in_specs=[pl.BlockSpec((_BM, D), lambda i: (i, 0)),
                  pl.BlockSpec((_BM, D), lambda i: (i, 0)),
                  pl.BlockSpec((_BM, D), lambda i: (i, 0)),
                  pl.BlockSpec((_BM, D), lambda i: (i, 0)),
                  pl.BlockSpec((1, D), lambda i: (0, 0)),
                  pl.BlockSpec((D, D), lambda i: (0, 0))],
        out_specs=pl.BlockSpec((_BM, D), lambda i: (i, 0)),
        out_shape=jax.ShapeDtypeStruct((NP, D), _f32),
    )(a0, a1, hs, dinv, b, w)


def _out_body(a0_ref, a1_ref, hs_ref, dinv_ref, b_ref, wc_ref, bc_ref,
              lo_ref, lp_ref):
    dinv = dinv_ref[...]
    z = dinv * (a0_ref[...] + a1_ref[...] + hs_ref[...]) + b_ref[...]
    logits = _dot(z, wc_ref[...]) + bc_ref[...]
    m = jnp.max(logits, axis=1, keepdims=True)
    lse = m + jnp.log(jnp.sum(jnp.exp(logits - m), axis=1, keepdims=True))
    lo_ref[...] = logits
    lp_ref[...] = logits - lse


def _tc_out(a0, a1, hs, dinv, b, wc, bc):
    return pl.pallas_call(
        _out_body,
        grid=(NP // _BM,),
        in_specs=[pl.BlockSpec((_BM, D), lambda i: (i, 0)),
                  pl.BlockSpec((_BM, D), lambda i: (i, 0)),
                  pl.BlockSpec((_BM, D), lambda i: (i, 0)),
                  pl.BlockSpec((_BM, D), lambda i: (i, 0)),
                  pl.BlockSpec((1, D), lambda i: (0, 0)),
                  pl.BlockSpec((D, D), lambda i: (0, 0)),
                  pl.BlockSpec((1, D), lambda i: (0, 0))],
        out_specs=[pl.BlockSpec((_BM, D), lambda i: (i, 0)),
                   pl.BlockSpec((_BM, D), lambda i: (i, 0))],
        out_shape=[jax.ShapeDtypeStruct((NP, D), _f32),
                   jax.ShapeDtypeStruct((NP, D), _f32)],
    )(a0, a1, hs, dinv, b, wc, bc)


def kernel(x, edge_index, W1, b1, W2, b2, Wc, bc):
    pad_rows = EROWS - E // CH
    src = jnp.pad(edge_index[0].reshape(E // CH, CH), ((0, pad_rows), (0, 0)))
    dst = jnp.pad(edge_index[1].reshape(E // CH, CH), ((0, pad_rows), (0, 0)))
    x_pad = jnp.pad(x, ((0, NP - N), (0, 0)))

    u1 = _tc_matmul(x_pad, W1)
    h0, h1, dstm = _sc_prep(src, dst)

    dinv, hs1 = _tc_scale(h0, h1, u1)
    a0, a1 = _sc_agg(hs1, src, dstm)
    hs2 = _tc_layer(a0, a1, hs1, dinv, b1.reshape(1, D), W2)
    c0, c1 = _sc_agg(hs2, src, dstm)

    wcp = jnp.pad(Wc, ((0, 0), (0, D - NC)))
    bcp = jnp.pad(bc, (0, D - NC), constant_values=-1e30).reshape(1, D)
    logits_pad, logp_pad = _tc_out(c0, c1, hs2, dinv, b2.reshape(1, D), wcp, bcp)
    return logits_pad[:N, :NC], logp_pad[:N, :NC]


# speedup vs baseline: 8.1483x; 1.0700x over previous
import functools

import jax
import jax.numpy as jnp
from jax import lax
from jax.experimental import pallas as pl
from jax.experimental.pallas import tpu as pltpu
from jax.experimental.pallas import tpu_sc as plsc

N = 10000
E = 320000
D = 128
NC = 40
NP = 10240
TRASH = N
NCORE = 2
NSUB = 16
NW = NCORE * NSUB
CH = 128
EROWS = 2560
NCH = EROWS // NW
SEG = 16
ROWS_PER_TILE = NP // NSUB

_mesh = plsc.VectorSubcoreMesh(core_axis_name="c", subcore_axis_name="s")

_f32 = jnp.float32
_i32 = jnp.int32


@functools.partial(
    pl.kernel,
    out_type=[
        jax.ShapeDtypeStruct((NP, D), _f32),
        jax.ShapeDtypeStruct((NP, D), _f32),
        jax.ShapeDtypeStruct((EROWS, CH), _i32),
    ],
    mesh=_mesh,
    scratch_types=[
        pltpu.VMEM((SEG, CH), _i32),
        pltpu.VMEM((SEG, CH), _i32),
        pltpu.VMEM((SEG, CH), _i32),
        pltpu.VMEM((CH, D), _f32),
        pltpu.VMEM_SHARED((NP, D), _f32),
    ],
)
def _sc_prep(src_hbm, dst_hbm, h0_hbm, h1_hbm, dstm_hbm,
             src_v, dst_v, dstm_v, buf_v, hist_sh):
    cid = lax.axis_index("c")
    sid = lax.axis_index("s")
    wid = sid * NCORE + cid
    base = wid * NCH

    z16 = jnp.zeros((16,), _f32)
    o16 = jnp.ones((16,), _f32)

    @pl.loop(0, CH)
    def _(r):
        @pl.loop(0, D // 16)
        def _(c):
            buf_v[r, pl.ds(c * 16, 16)] = z16

    @pl.loop(0, ROWS_PER_TILE // CH)
    def _(k):
        pltpu.sync_copy(buf_v, hist_sh.at[pl.ds(sid * ROWS_PER_TILE + k * CH, CH)])

    @pl.loop(0, CH)
    def _(r):
        @pl.loop(0, D // 16)
        def _(c):
            buf_v[r, pl.ds(c * 16, 16)] = o16

    plsc.subcore_barrier()

    for g in range(NCH // SEG):
        gb = base + g * SEG
        pltpu.sync_copy(src_hbm.at[pl.ds(gb, SEG)], src_v)
        pltpu.sync_copy(dst_hbm.at[pl.ds(gb, SEG)], dst_v)

        @pl.loop(0, SEG)
        def _(r):
            @pl.loop(0, CH // 16)
            def _(c):
                s = src_v[r, pl.ds(c * 16, 16)]
                dv = dst_v[r, pl.ds(c * 16, 16)]
                dstm_v[r, pl.ds(c * 16, 16)] = jnp.where(s == dv, TRASH, dv)

        pltpu.sync_copy(dstm_v, dstm_hbm.at[pl.ds(gb, SEG)])

        @pl.loop(0, SEG)
        def _(r):
            pltpu.sync_copy(buf_v, hist_sh.at[dstm_v.at[r]], add=True)

    plsc.subcore_barrier()

    @pl.when(cid == 0)
    def _():
        pltpu.sync_copy(hist_sh.at[pl.ds(sid * ROWS_PER_TILE, ROWS_PER_TILE)],
                        h0_hbm.at[pl.ds(sid * ROWS_PER_TILE, ROWS_PER_TILE)])

    @pl.when(cid == 1)
    def _():
        pltpu.sync_copy(hist_sh.at[pl.ds(sid * ROWS_PER_TILE, ROWS_PER_TILE)],
                        h1_hbm.at[pl.ds(sid * ROWS_PER_TILE, ROWS_PER_TILE)])


@functools.partial(
    pl.kernel,
    out_type=[
        jax.ShapeDtypeStruct((NP, D), _f32),
        jax.ShapeDtypeStruct((NP, D), _f32),
    ],
    mesh=_mesh,
    scratch_types=[
        pltpu.VMEM((SEG, CH), _i32),
        pltpu.VMEM((SEG, CH), _i32),
        pltpu.VMEM((CH, D), _f32),
        pltpu.VMEM((CH, D), _f32),
        pltpu.VMEM_SHARED((NP, D), _f32),
        pltpu.SemaphoreType.DMA,
        pltpu.SemaphoreType.DMA,
    ],
)
def _sc_agg(hs_hbm, src_hbm, dstm_hbm, a0_hbm, a1_hbm,
            src_v, dstm_v, rows0, rows1, acc_sh, gsem0, gsem1):
    cid = lax.axis_index("c")
    sid = lax.axis_index("s")
    wid = sid * NCORE + cid
    base = wid * NCH

    z16 = jnp.zeros((16,), _f32)

    @pl.loop(0, CH)
    def _(r):
        @pl.loop(0, D // 16)
        def _(c):
            rows0[r, pl.ds(c * 16, 16)] = z16

    @pl.loop(0, ROWS_PER_TILE // CH)
    def _(k):
        pltpu.sync_copy(rows0, acc_sh.at[pl.ds(sid * ROWS_PER_TILE + k * CH, CH)])

    plsc.subcore_barrier()

    for g in range(NCH // SEG):
        gb = base + g * SEG
        pltpu.sync_copy(src_hbm.at[pl.ds(gb, SEG)], src_v)
        pltpu.sync_copy(dstm_hbm.at[pl.ds(gb, SEG)], dstm_v)

        pltpu.sync_copy(hs_hbm.at[src_v.at[0]], rows0)

        @pl.loop(0, SEG, step=2)
        def _(i):
            c1 = pltpu.async_copy(hs_hbm.at[src_v.at[i + 1]], rows1, gsem1)
            pltpu.sync_copy(rows0, acc_sh.at[dstm_v.at[i]], add=True)
            c1.wait()

            @pl.when(i + 2 < SEG)
            def _():
                c2 = pltpu.async_copy(hs_hbm.at[src_v.at[i + 2]], rows0, gsem0)
                pltpu.sync_copy(rows1, acc_sh.at[dstm_v.at[i + 1]], add=True)
                c2.wait()

            @pl.when(i + 2 >= SEG)
            def _():
                pltpu.sync_copy(rows1, acc_sh.at[dstm_v.at[i + 1]], add=True)

    plsc.subcore_barrier()

    @pl.when(cid == 0)
    def _():
        pltpu.sync_copy(acc_sh.at[pl.ds(sid * ROWS_PER_TILE, ROWS_PER_TILE)],
                        a0_hbm.at[pl.ds(sid * ROWS_PER_TILE, ROWS_PER_TILE)])

    @pl.when(cid == 1)
    def _():
        pltpu.sync_copy(acc_sh.at[pl.ds(sid * ROWS_PER_TILE, ROWS_PER_TILE)],
                        a1_hbm.at[pl.ds(sid * ROWS_PER_TILE, ROWS_PER_TILE)])


_BM = 1024


def _dot(a, b):
    return jnp.dot(a, b, preferred_element_type=_f32,
                   precision=jax.lax.Precision.HIGHEST)


def _mm_body(x_ref, w_ref, o_ref):
    o_ref[...] = _dot(x_ref[...], w_ref[...])


def _tc_matmul(x, w):
    return pl.pallas_call(
        _mm_body,
        grid=(NP // _BM,),
        in_specs=[pl.BlockSpec((_BM, D), lambda i: (i, 0)),
                  pl.BlockSpec((D, D), lambda i: (0, 0))],
        out_specs=pl.BlockSpec((_BM, D), lambda i: (i, 0)),
        out_shape=jax.ShapeDtypeStruct((NP, D), _f32),
    )(x, w)


def _scale_body(h0_ref, h1_ref, u_ref, dinv_ref, hs_ref):
    deg = h0_ref[...] + h1_ref[...] + 1.0
    dinv = lax.rsqrt(deg)
    dinv_ref[...] = dinv
    hs_ref[...] = dinv * u_ref[...]


def _tc_scale(h0, h1, u):
    return pl.pallas_call(
        _scale_body,
        grid=(NP // _BM,),
        in_specs=[pl.BlockSpec((_BM, D), lambda i: (i, 0)),
                  pl.BlockSpec((_BM, D), lambda i: (i, 0)),
                  pl.BlockSpec((_BM, D), lambda i: (i, 0))],
        out_specs=[pl.BlockSpec((_BM, D), lambda i: (i, 0)),
                   pl.BlockSpec((_BM, D), lambda i: (i, 0))],
        out_shape=[jax.ShapeDtypeStruct((NP, D), _f32),
                   jax.ShapeDtypeStruct((NP, D), _f32)],
    )(h0, h1, u)


def _layer_body(a0_ref, a1_ref, hs_ref, dinv_ref, b_ref, w_ref, o_ref):
    dinv = dinv_ref[...]
    z = dinv * (a0_ref[...] + a1_ref[...] + hs_ref[...]) + b_ref[...]
    z = jnp.maximum(z, 0.0)
    o_ref[...] = dinv * _dot(z, w_ref[...])


def _tc_layer(a0, a1, hs, dinv, b, w):
    return pl.pallas_call(
        _layer_body,
        grid=(NP // _BM,),
        in_specs=[pl.BlockSpec((_BM, D), lambda i: (i, 0)),
                  pl.BlockSpec((_BM, D), lambda i: (i, 0)),
                  pl.BlockSpec((_BM, D), lambda i: (i, 0)),
                  pl.BlockSpec((_BM, D), lambda i: (i, 0)),
                  pl.BlockSpec((1, D), lambda i: (0, 0)),
                  pl.BlockSpec((D, D), lambda i: (0, 0))],
        out_specs=pl.BlockSpec((_BM, D), lambda i: (i, 0)),
        out_shape=jax.ShapeDtypeStruct((NP, D), _f32),
    )(a0, a1, hs, dinv, b, w)


def _out_body(a0_ref, a1_ref, hs_ref, dinv_ref, b_ref, wc_ref, bc_ref,
              lo_ref, lp_ref):
    dinv = dinv_ref[...]
    z = dinv * (a0_ref[...] + a1_ref[...] + hs_ref[...]) + b_ref[...]
    logits = _dot(z, wc_ref[...]) + bc_ref[...]
    m = jnp.max(logits, axis=1, keepdims=True)
    lse = m + jnp.log(jnp.sum(jnp.exp(logits - m), axis=1, keepdims=True))
    lo_ref[...] = logits
    lp_ref[...] = logits - lse


def _tc_out(a0, a1, hs, dinv, b, wc, bc):
    return pl.pallas_call(
        _out_body,
        grid=(NP // _BM,),
        in_specs=[pl.BlockSpec((_BM, D), lambda i: (i, 0)),
                  pl.BlockSpec((_BM, D), lambda i: (i, 0)),
                  pl.BlockSpec((_BM, D), lambda i: (i, 0)),
                  pl.BlockSpec((_BM, D), lambda i: (i, 0)),
                  pl.BlockSpec((1, D), lambda i: (0, 0)),
                  pl.BlockSpec((D, D), lambda i: (0, 0)),
                  pl.BlockSpec((1, D), lambda i: (0, 0))],
        out_specs=[pl.BlockSpec((_BM, D), lambda i: (i, 0)),
                   pl.BlockSpec((_BM, D), lambda i: (i, 0))],
        out_shape=[jax.ShapeDtypeStruct((NP, D), _f32),
                   jax.ShapeDtypeStruct((NP, D), _f32)],
    )(a0, a1, hs, dinv, b, wc, bc)


def kernel(x, edge_index, W1, b1, W2, b2, Wc, bc):
    pad_rows = EROWS - E // CH
    src = jnp.pad(edge_index[0].reshape(E // CH, CH), ((0, pad_rows), (0, 0)))
    dst = jnp.pad(edge_index[1].reshape(E // CH, CH), ((0, pad_rows), (0, 0)))
    x_pad = jnp.pad(x, ((0, NP - N), (0, 0)))

    u1 = _tc_matmul(x_pad, W1)
    h0, h1, dstm = _sc_prep(src, dst)

    dinv, hs1 = _tc_scale(h0, h1, u1)
    a0, a1 = _sc_agg(hs1, src, dstm)
    hs2 = _tc_layer(a0, a1, hs1, dinv, b1.reshape(1, D), W2)
    c0, c1 = _sc_agg(hs2, src, dstm)

    wcp = jnp.pad(Wc, ((0, 0), (0, D - NC)))
    bcp = jnp.pad(bc, (0, D - NC), constant_values=-1e30).reshape(1, D)
    logits_pad, logp_pad = _tc_out(c0, c1, hs2, dinv, b2.reshape(1, D), wcp, bcp)
    return logits_pad[:N, :NC], logp_pad[:N, :NC]


# speedup vs baseline: 8.3169x; 1.0207x over previous
import functools

import jax
import jax.numpy as jnp
from jax import lax
from jax.experimental import pallas as pl
from jax.experimental.pallas import tpu as pltpu
from jax.experimental.pallas import tpu_sc as plsc

N = 10000
E = 320000
D = 128
NC = 40
NP = 10240
TRASH = N
NCORE = 2
NSUB = 16
NW = NCORE * NSUB
CH = 128
EROWS = 2560
NCH = EROWS // NW
SEG = 16
ROWS_PER_TILE = NP // NSUB

_mesh = plsc.VectorSubcoreMesh(core_axis_name="c", subcore_axis_name="s")

_f32 = jnp.float32
_i32 = jnp.int32


@functools.partial(
    pl.kernel,
    out_type=[
        jax.ShapeDtypeStruct((NP, D), _f32),
        jax.ShapeDtypeStruct((NP, D), _f32),
        jax.ShapeDtypeStruct((EROWS, CH), _i32),
    ],
    mesh=_mesh,
    scratch_types=[
        pltpu.VMEM((SEG, CH), _i32),
        pltpu.VMEM((SEG, CH), _i32),
        pltpu.VMEM((SEG, CH), _i32),
        pltpu.VMEM((CH, D), _f32),
        pltpu.VMEM_SHARED((NP, D), _f32),
    ],
)
def _sc_prep(src_hbm, dst_hbm, h0_hbm, h1_hbm, dstm_hbm,
             src_v, dst_v, dstm_v, buf_v, hist_sh):
    cid = lax.axis_index("c")
    sid = lax.axis_index("s")
    wid = sid * NCORE + cid
    base = wid * NCH

    z16 = jnp.zeros((16,), _f32)
    o16 = jnp.ones((16,), _f32)

    @pl.loop(0, CH)
    def _(r):
        @pl.loop(0, D // 16)
        def _(c):
            buf_v[r, pl.ds(c * 16, 16)] = z16

    @pl.loop(0, ROWS_PER_TILE // CH)
    def _(k):
        pltpu.sync_copy(buf_v, hist_sh.at[pl.ds(sid * ROWS_PER_TILE + k * CH, CH)])

    @pl.loop(0, CH)
    def _(r):
        @pl.loop(0, D // 16)
        def _(c):
            buf_v[r, pl.ds(c * 16, 16)] = o16

    plsc.subcore_barrier()

    for g in range(NCH // SEG):
        gb = base + g * SEG
        pltpu.sync_copy(src_hbm.at[pl.ds(gb, SEG)], src_v)
        pltpu.sync_copy(dst_hbm.at[pl.ds(gb, SEG)], dst_v)

        @pl.loop(0, SEG)
        def _(r):
            @pl.loop(0, CH // 16)
            def _(c):
                s = src_v[r, pl.ds(c * 16, 16)]
                dv = dst_v[r, pl.ds(c * 16, 16)]
                dstm_v[r, pl.ds(c * 16, 16)] = jnp.where(s == dv, TRASH, dv)

        pltpu.sync_copy(dstm_v, dstm_hbm.at[pl.ds(gb, SEG)])

        @pl.loop(0, SEG)
        def _(r):
            pltpu.sync_copy(buf_v, hist_sh.at[dstm_v.at[r]], add=True)

    plsc.subcore_barrier()

    @pl.when(cid == 0)
    def _():
        pltpu.sync_copy(hist_sh.at[pl.ds(sid * ROWS_PER_TILE, ROWS_PER_TILE)],
                        h0_hbm.at[pl.ds(sid * ROWS_PER_TILE, ROWS_PER_TILE)])

    @pl.when(cid == 1)
    def _():
        pltpu.sync_copy(hist_sh.at[pl.ds(sid * ROWS_PER_TILE, ROWS_PER_TILE)],
                        h1_hbm.at[pl.ds(sid * ROWS_PER_TILE, ROWS_PER_TILE)])


@functools.partial(
    pl.kernel,
    out_type=[
        jax.ShapeDtypeStruct((NP, D), _f32),
        jax.ShapeDtypeStruct((NP, D), _f32),
    ],
    mesh=_mesh,
    scratch_types=[
        pltpu.VMEM((SEG, CH), _i32),
        pltpu.VMEM((SEG, CH), _i32),
        pltpu.VMEM((CH, D), _f32),
        pltpu.VMEM((CH, D), _f32),
        pltpu.VMEM_SHARED((NP, D), _f32),
        pltpu.SemaphoreType.DMA,
        pltpu.SemaphoreType.DMA,
    ],
)
def _sc_agg(hs_hbm, src_hbm, dstm_hbm, a0_hbm, a1_hbm,
            src_v, dstm_v, rows0, rows1, acc_sh, gsem0, gsem1):
    cid = lax.axis_index("c")
    sid = lax.axis_index("s")
    wid = sid * NCORE + cid
    base = wid * NCH

    z16 = jnp.zeros((16,), _f32)

    @pl.loop(0, CH)
    def _(r):
        @pl.loop(0, D // 16)
        def _(c):
            rows0[r, pl.ds(c * 16, 16)] = z16

    @pl.loop(0, ROWS_PER_TILE // CH)
    def _(k):
        pltpu.sync_copy(rows0, acc_sh.at[pl.ds(sid * ROWS_PER_TILE + k * CH, CH)])

    plsc.subcore_barrier()

    for g in range(NCH // SEG):
        gb = base + g * SEG
        pltpu.sync_copy(src_hbm.at[pl.ds(gb, SEG)], src_v)
        pltpu.sync_copy(dstm_hbm.at[pl.ds(gb, SEG)], dstm_v)

        pltpu.async_copy(hs_hbm.at[src_v.at[0]], rows0, gsem0)
        pltpu.async_copy(hs_hbm.at[src_v.at[1]], rows1, gsem1)

        @pl.loop(0, SEG - 2, step=2)
        def _(i):
            pltpu.make_async_copy(hs_hbm.at[src_v.at[0]], rows0, gsem0).wait()
            pltpu.sync_copy(rows0, acc_sh.at[dstm_v.at[i]], add=True)
            pltpu.async_copy(hs_hbm.at[src_v.at[i + 2]], rows0, gsem0)
            pltpu.make_async_copy(hs_hbm.at[src_v.at[0]], rows1, gsem1).wait()
            pltpu.sync_copy(rows1, acc_sh.at[dstm_v.at[i + 1]], add=True)
            pltpu.async_copy(hs_hbm.at[src_v.at[i + 3]], rows1, gsem1)

        pltpu.make_async_copy(hs_hbm.at[src_v.at[0]], rows0, gsem0).wait()
        pltpu.sync_copy(rows0, acc_sh.at[dstm_v.at[SEG - 2]], add=True)
        pltpu.make_async_copy(hs_hbm.at[src_v.at[0]], rows1, gsem1).wait()
        pltpu.sync_copy(rows1, acc_sh.at[dstm_v.at[SEG - 1]], add=True)

    plsc.subcore_barrier()

    @pl.when(cid == 0)
    def _():
        pltpu.sync_copy(acc_sh.at[pl.ds(sid * ROWS_PER_TILE, ROWS_PER_TILE)],
                        a0_hbm.at[pl.ds(sid * ROWS_PER_TILE, ROWS_PER_TILE)])

    @pl.when(cid == 1)
    def _():
        pltpu.sync_copy(acc_sh.at[pl.ds(sid * ROWS_PER_TILE, ROWS_PER_TILE)],
                        a1_hbm.at[pl.ds(sid * ROWS_PER_TILE, ROWS_PER_TILE)])


_BM = 1024


def _dot(a, b):
    return jnp.dot(a, b, preferred_element_type=_f32,
                   precision=jax.lax.Precision.HIGHEST)


def _mm_body(x_ref, w_ref, o_ref):
    o_ref[...] = _dot(x_ref[...], w_ref[...])


def _tc_matmul(x, w):
    return pl.pallas_call(
        _mm_body,
        grid=(NP // _BM,),
        in_specs=[pl.BlockSpec((_BM, D), lambda i: (i, 0)),
                  pl.BlockSpec((D, D), lambda i: (0, 0))],
        out_specs=pl.BlockSpec((_BM, D), lambda i: (i, 0)),
        out_shape=jax.ShapeDtypeStruct((NP, D), _f32),
    )(x, w)


def _scale_body(h0_ref, h1_ref, u_ref, dinv_ref, hs_ref):
    deg = h0_ref[...] + h1_ref[...] + 1.0
    dinv = lax.rsqrt(deg)
    dinv_ref[...] = dinv
    hs_ref[...] = dinv * u_ref[...]


def _tc_scale(h0, h1, u):
    return pl.pallas_call(
        _scale_body,
        grid=(NP // _BM,),
        in_specs=[pl.BlockSpec((_BM, D), lambda i: (i, 0)),
                  pl.BlockSpec((_BM, D), lambda i: (i, 0)),
                  pl.BlockSpec((_BM, D), lambda i: (i, 0))],
        out_specs=[pl.BlockSpec((_BM, D), lambda i: (i, 0)),
                   pl.BlockSpec((_BM, D), lambda i: (i, 0))],
        out_shape=[jax.ShapeDtypeStruct((NP, D), _f32),
                   jax.ShapeDtypeStruct((NP, D), _f32)],
    )(h0, h1, u)


def _layer_body(a0_ref, a1_ref, hs_ref, dinv_ref, b_ref, w_ref, o_ref):
    dinv = dinv_ref[...]
    z = dinv * (a0_ref[...] + a1_ref[...] + hs_ref[...]) + b_ref[...]
    z = jnp.maximum(z, 0.0)
    o_ref[...] = dinv * _dot(z, w_ref[...])


def _tc_layer(a0, a1, hs, dinv, b, w):
    return pl.pallas_call(
        _layer_body,
        grid=(NP // _BM,),
        in_specs=[pl.BlockSpec((_BM, D), lambda i: (i, 0)),
                  pl.BlockSpec((_BM, D), lambda i: (i, 0)),
                  pl.BlockSpec((_BM, D), lambda i: (i, 0)),
                  pl.BlockSpec((_BM, D), lambda i: (i, 0)),
                  pl.BlockSpec((1, D), lambda i: (0, 0)),
                  pl.BlockSpec((D, D), lambda i: (0, 0))],
        out_specs=pl.BlockSpec((_BM, D), lambda i: (i, 0)),
        out_shape=jax.ShapeDtypeStruct((NP, D), _f32),
    )(a0, a1, hs, dinv, b, w)


def _out_body(a0_ref, a1_ref, hs_ref, dinv_ref, b_ref, wc_ref, bc_ref,
              lo_ref, lp_ref):
    dinv = dinv_ref[...]
    z = dinv * (a0_ref[...] + a1_ref[...] + hs_ref[...]) + b_ref[...]
    logits = _dot(z, wc_ref[...]) + bc_ref[...]
    m = jnp.max(logits, axis=1, keepdims=True)
    lse = m + jnp.log(jnp.sum(jnp.exp(logits - m), axis=1, keepdims=True))
    lo_ref[...] = logits
    lp_ref[...] = logits - lse


def _tc_out(a0, a1, hs, dinv, b, wc, bc):
    return pl.pallas_call(
        _out_body,
        grid=(NP // _BM,),
        in_specs=[pl.BlockSpec((_BM, D), lambda i: (i, 0)),
                  pl.BlockSpec((_BM, D), lambda i: (i, 0)),
                  pl.BlockSpec((_BM, D), lambda i: (i, 0)),
                  pl.BlockSpec((_BM, D), lambda i: (i, 0)),
                  pl.BlockSpec((1, D), lambda i: (0, 0)),
                  pl.BlockSpec((D, D), lambda i: (0, 0)),
                  pl.BlockSpec((1, D), lambda i: (0, 0))],
        out_specs=[pl.BlockSpec((_BM, D), lambda i: (i, 0)),
                   pl.BlockSpec((_BM, D), lambda i: (i, 0))],
        out_shape=[jax.ShapeDtypeStruct((NP, D), _f32),
                   jax.ShapeDtypeStruct((NP, D), _f32)],
    )(a0, a1, hs, dinv, b, wc, bc)


def kernel(x, edge_index, W1, b1, W2, b2, Wc, bc):
    pad_rows = EROWS - E // CH
    src = jnp.pad(edge_index[0].reshape(E // CH, CH), ((0, pad_rows), (0, 0)))
    dst = jnp.pad(edge_index[1].reshape(E // CH, CH), ((0, pad_rows), (0, 0)))
    x_pad = jnp.pad(x, ((0, NP - N), (0, 0)))

    u1 = _tc_matmul(x_pad, W1)
    h0, h1, dstm = _sc_prep(src, dst)

    dinv, hs1 = _tc_scale(h0, h1, u1)
    a0, a1 = _sc_agg(hs1, src, dstm)
    hs2 = _tc_layer(a0, a1, hs1, dinv, b1.reshape(1, D), W2)
    c0, c1 = _sc_agg(hs2, src, dstm)

    wcp = jnp.pad(Wc, ((0, 0), (0, D - NC)))
    bcp = jnp.pad(bc, (0, D - NC), constant_values=-1e30).reshape(1, D)
    logits_pad, logp_pad = _tc_out(c0, c1, hs2, dinv, b2.reshape(1, D), wcp, bcp)
    return logits_pad[:N, :NC], logp_pad[:N, :NC]
